# baseline (speedup 1.0000x reference)
import functools

import jax
import jax.numpy as jnp
from jax import lax
from jax.experimental import pallas as pl
from jax.experimental.pallas import tpu as pltpu
from jax.experimental.pallas import tpu_sc as plsc

_B = 16384
_DS = (32, 32, 16, 16)
_COLS = (0, 32, 64, 80)
_DTOT = 96


def _build():
    info = plsc.get_sparse_core_info()
    nc, ns = info.num_cores, info.num_subcores
    nw = nc * ns
    bpw = _B // nw

    mesh = plsc.VectorSubcoreMesh(core_axis_name="c", subcore_axis_name="s")

    @functools.partial(
        pl.kernel,
        mesh=mesh,
        out_type=jax.ShapeDtypeStruct((_B * _DTOT,), jnp.float32),
        compiler_params=pltpu.CompilerParams(
            use_tc_tiling_on_sc=False, needs_layout_passes=False
        ),
        scratch_types=[
            pltpu.VMEM((4, bpw), jnp.int32),
            pltpu.VMEM((_DS[0], bpw), jnp.float32),
            pltpu.VMEM((_DS[1], bpw), jnp.float32),
            pltpu.VMEM((_DS[2], bpw), jnp.float32),
            pltpu.VMEM((_DS[3], bpw), jnp.float32),
            pltpu.VMEM((bpw * _DTOT,), jnp.float32),
            pltpu.SemaphoreType.DMA,
            pltpu.SemaphoreType.DMA,
            pltpu.SemaphoreType.DMA,
            pltpu.SemaphoreType.DMA,
        ],
    )
    def emb_kernel(u_hbm, i_hbm, c_hbm, b_hbm, wut, wit, wct, wbt,
                   out_hbm, idx_v, g0, g1, g2, g3, rows_v, s0, s1, s2, s3):
        wid = lax.axis_index("s") * nc + lax.axis_index("c")
        base = wid * bpw
        idx_refs = (u_hbm, i_hbm, c_hbm, b_hbm)
        tables = (wut, wit, wct, wbt)
        gbufs = (g0, g1, g2, g3)
        sems = (s0, s1, s2, s3)
        copies = []
        for t in range(4):
            pltpu.sync_copy(idx_refs[t].at[pl.ds(base, bpw)], idx_v.at[t])
            for c in range(_DS[t]):
                copies.append(
                    pltpu.async_copy(
                        tables[t].at[c].at[idx_v.at[t]],
                        gbufs[t].at[c],
                        sems[t],
                    )
                )
        for cp in copies:
            cp.wait()

        def body(g, idxvec):
            for t in range(4):
                for c in range(_DS[t]):
                    vals = gbufs[t][c, pl.ds(g * 16, 16)]
                    plsc.store_scatter(
                        rows_v, [idxvec + (_COLS[t] + c)], vals
                    )
            return idxvec + (16 * _DTOT)

        lax.fori_loop(0, bpw // 16, body,
                      lax.iota(jnp.int32, 16) * _DTOT)
        pltpu.sync_copy(rows_v, out_hbm.at[pl.ds(base * _DTOT, bpw * _DTOT)])

    return emb_kernel


_emb_kernel = _build()


def kernel(user_id, item_id, category, brand,
           W_user_id, W_item_id, W_category, W_brand):
    out = _emb_kernel(user_id, item_id, category, brand,
                      W_user_id.T, W_item_id.T, W_category.T, W_brand.T)
    return out.reshape(_B, _DTOT)


# speedup vs baseline: 6.6647x; 6.6647x over previous
import functools

import jax
import jax.numpy as jnp
from jax import lax
from jax.experimental import pallas as pl
from jax.experimental.pallas import tpu as pltpu
from jax.experimental.pallas import tpu_sc as plsc

_B = 16384
_DS = (32, 32, 16, 16)
_COLS = (0, 32, 64, 80)
_DTOT = 96
_CH = 16


def _build():
    info = plsc.get_sparse_core_info()
    nc, ns = info.num_cores, info.num_subcores
    nw = nc * ns
    bpw = _B // nw

    mesh = plsc.VectorSubcoreMesh(core_axis_name="c", subcore_axis_name="s")

    @functools.partial(
        pl.kernel,
        mesh=mesh,
        out_type=jax.ShapeDtypeStruct((_B * _DTOT,), jnp.float32),
        scratch_types=[
            pltpu.VMEM((4, bpw), jnp.int32),
            pltpu.VMEM((_CH, 8, _DS[0]), jnp.float32),
            pltpu.VMEM((_CH, 8, _DS[1]), jnp.float32),
            pltpu.VMEM((_CH, 8, _DS[2]), jnp.float32),
            pltpu.VMEM((_CH, 8, _DS[3]), jnp.float32),
            pltpu.VMEM((bpw * _DTOT,), jnp.float32),
            pltpu.SemaphoreType.DMA,
        ],
    )
    def emb_kernel(u_hbm, i_hbm, c_hbm, b_hbm, wu, wi, wc, wb,
                   out_hbm, idx_v, sl0, sl1, sl2, sl3, rows_v, sem):
        wid = lax.axis_index("s") * nc + lax.axis_index("c")
        base = wid * bpw
        idx_refs = (u_hbm, i_hbm, c_hbm, b_hbm)
        tables = (wu, wi, wc, wb)
        slots = (sl0, sl1, sl2, sl3)
        for t in range(4):
            pltpu.sync_copy(idx_refs[t].at[pl.ds(base, bpw)], idx_v.at[t])

        def chunk(i, carry):
            vs = [idx_v[t, pl.ds(i * _CH, _CH)] for t in range(4)]
            cps = []
            for t in range(4):
                for j in range(_CH):
                    r = vs[t][j]
                    t_off = pl.multiple_of((r // 8) * 8, 8)
                    cps.append(
                        pltpu.async_copy(
                            tables[t].at[pl.ds(t_off, 8)], slots[t].at[j], sem
                        )
                    )
            for cp in cps:
                cp.wait()
            for t in range(4):
                for j in range(_CH):
                    r = vs[t][j]
                    row = lax.rem(r, 8)
                    for c in range(0, _DS[t], 16):
                        rows_v[
                            pl.ds((i * _CH + j) * _DTOT + _COLS[t] + c, 16)
                        ] = slots[t][j, row, pl.ds(c, 16)]
            return carry

        lax.fori_loop(0, bpw // _CH, chunk, 0)
        pltpu.sync_copy(rows_v, out_hbm.at[pl.ds(base * _DTOT, bpw * _DTOT)])

    return emb_kernel


_emb_kernel = _build()


def kernel(user_id, item_id, category, brand,
           W_user_id, W_item_id, W_category, W_brand):
    out = _emb_kernel(user_id, item_id, category, brand,
                      W_user_id, W_item_id, W_category, W_brand)
    return out.reshape(_B, _DTOT)


# speedup vs baseline: 6.6721x; 1.0011x over previous
import functools

import jax
import jax.numpy as jnp
from jax import lax
from jax.experimental import pallas as pl
from jax.experimental.pallas import tpu as pltpu
from jax.experimental.pallas import tpu_sc as plsc

_B = 16384
_DS = (32, 32, 16, 16)
_COLS = (0, 32, 64, 80)
_DTOT = 96
_CH = 8
_NB = 2


def _build():
    info = plsc.get_sparse_core_info()
    nc, ns = info.num_cores, info.num_subcores
    nw = nc * ns
    bpw = _B // nw

    mesh = plsc.VectorSubcoreMesh(core_axis_name="c", subcore_axis_name="s")

    @functools.partial(
        pl.kernel,
        mesh=mesh,
        out_type=jax.ShapeDtypeStruct((_B * _DTOT,), jnp.float32),
        scratch_types=[
            pltpu.VMEM((4, bpw), jnp.int32),
            pltpu.VMEM((_NB, _CH, 8, _DS[0]), jnp.float32),
            pltpu.VMEM((_NB, _CH, 8, _DS[1]), jnp.float32),
            pltpu.VMEM((_NB, _CH, 8, _DS[2]), jnp.float32),
            pltpu.VMEM((_NB, _CH, 8, _DS[3]), jnp.float32),
            pltpu.VMEM((bpw * _DTOT,), jnp.float32),
            pltpu.SemaphoreType.DMA,
        ],
    )
    def emb_kernel(u_hbm, i_hbm, c_hbm, b_hbm, wu, wi, wc, wb,
                   out_hbm, idx_v, sl0, sl1, sl2, sl3, rows_v, sem):
        wid = lax.axis_index("s") * nc + lax.axis_index("c")
        base = wid * bpw
        idx_refs = (u_hbm, i_hbm, c_hbm, b_hbm)
        tables = (wu, wi, wc, wb)
        slots = (sl0, sl1, sl2, sl3)
        for t in range(4):
            pltpu.sync_copy(idx_refs[t].at[pl.ds(base, bpw)], idx_v.at[t])

        def issue(i, b, vhalf):
            cps = []
            for t in range(4):
                for j in range(_CH):
                    r = vhalf[t][j]
                    t_off = pl.multiple_of((r // 8) * 8, 8)
                    cps.append(
                        pltpu.async_copy(
                            tables[t].at[pl.ds(t_off, 8)],
                            slots[t].at[b, j],
                            sem,
                        )
                    )
            return cps

        def fixup(i, b, vhalf, cps):
            for cp in cps:
                cp.wait()
            for t in range(4):
                for j in range(_CH):
                    r = vhalf[t][j]
                    row = lax.rem(r, 8)
                    for c in range(0, _DS[t], 16):
                        rows_v[
                            pl.ds((i * _CH + j) * _DTOT + _COLS[t] + c, 16)
                        ] = slots[t][b, j, row, pl.ds(c, 16)]

        def pair(p, carry):
            i0 = p * _NB
            vs = [idx_v[t, pl.ds(i0 * _CH, 16)] for t in range(4)]
            v0 = [[vs[t][j] for j in range(8)] for t in range(4)]
            v1 = [[vs[t][j + 8] for j in range(8)] for t in range(4)]
            c0 = issue(i0, 0, v0)
            c1 = issue(i0 + 1, 1, v1)
            fixup(i0, 0, v0, c0)
            fixup(i0 + 1, 1, v1, c1)
            return carry

        lax.fori_loop(0, bpw // _CH // _NB, pair, 0)
        pltpu.sync_copy(rows_v, out_hbm.at[pl.ds(base * _DTOT, bpw * _DTOT)])

    return emb_kernel


_emb_kernel = _build()


def kernel(user_id, item_id, category, brand,
           W_user_id, W_item_id, W_category, W_brand):
    out = _emb_kernel(user_id, item_id, category, brand,
                      W_user_id, W_item_id, W_category, W_brand)
    return out.reshape(_B, _DTOT)


# speedup vs baseline: 6.9749x; 1.0454x over previous
import functools

import jax
import jax.numpy as jnp
from jax import lax
from jax.experimental import pallas as pl
from jax.experimental.pallas import tpu as pltpu
from jax.experimental.pallas import tpu_sc as plsc

_B = 16384
_DTOT = 96
_CH = 8
_NB = 2


def _build(ds, cols):
    info = plsc.get_sparse_core_info()
    nc, ns = info.num_cores, info.num_subcores
    nw = nc * ns
    bpw = _B // nw
    nt = len(ds)
    zero_cols = sorted(
        set(range(0, _DTOT, 16))
        - {c0 + c for c0, d in zip(cols, ds) for c in range(0, d, 16)}
    )

    mesh = plsc.VectorSubcoreMesh(core_axis_name="c", subcore_axis_name="s")

    @functools.partial(
        pl.kernel,
        mesh=mesh,
        out_type=jax.ShapeDtypeStruct((_B * _DTOT,), jnp.float32),
        scratch_types=[
            pltpu.VMEM((4, bpw), jnp.int32),
            *[pltpu.VMEM((_NB, _CH, 8, d), jnp.float32) for d in ds],
            pltpu.VMEM((bpw * _DTOT,), jnp.float32),
            pltpu.SemaphoreType.DMA,
        ],
    )
    def emb_kernel(*refs):
        idx_refs = refs[:nt]
        tables = refs[nt:2 * nt]
        out_hbm = refs[2 * nt]
        idx_v = refs[2 * nt + 1]
        slots = refs[2 * nt + 2:2 * nt + 2 + nt]
        rows_v = refs[2 * nt + 2 + nt]
        sem = refs[2 * nt + 3 + nt]

        wid = lax.axis_index("s") * nc + lax.axis_index("c")
        base = wid * bpw
        for t in range(nt):
            pltpu.sync_copy(idx_refs[t].at[pl.ds(base, bpw)], idx_v.at[t])

        zvec = jnp.zeros((16,), jnp.float32)

        def issue(b, vhalf):
            cps = []
            for t in range(nt):
                for j in range(_CH):
                    r = vhalf[t][j]
                    t_off = pl.multiple_of((r // 8) * 8, 8)
                    cps.append(
                        pltpu.async_copy(
                            tables[t].at[pl.ds(t_off, 8)],
                            slots[t].at[b, j],
                            sem,
                        )
                    )
            return cps

        def fixup(i, b, vhalf, cps):
            for cp in cps:
                cp.wait()
            for t in range(nt):
                for j in range(_CH):
                    r = vhalf[t][j]
                    row = lax.rem(r, 8)
                    for c in range(0, ds[t], 16):
                        rows_v[
                            pl.ds((i * _CH + j) * _DTOT + cols[t] + c, 16)
                        ] = slots[t][b, j, row, pl.ds(c, 16)]
                for j in range(_CH):
                    for zc in zero_cols:
                        rows_v[pl.ds((i * _CH + j) * _DTOT + zc, 16)] = zvec

        def pair(p, carry):
            i0 = p * _NB
            vs = [idx_v[t, pl.ds(i0 * _CH, 16)] for t in range(nt)]
            v0 = [[vs[t][j] for j in range(_CH)] for t in range(nt)]
            v1 = [[vs[t][j + _CH] for j in range(_CH)] for t in range(nt)]
            c0 = issue(0, v0)
            c1 = issue(1, v1)
            fixup(i0, 0, v0, c0)
            fixup(i0 + 1, 1, v1, c1)
            return carry

        lax.fori_loop(0, bpw // _CH // _NB, pair, 0)
        pltpu.sync_copy(rows_v, out_hbm.at[pl.ds(base * _DTOT, bpw * _DTOT)])

    return emb_kernel


_kernel_a = _build(ds=(32, 16, 16), cols=(0, 64, 80))
_kernel_b = _build(ds=(32,), cols=(32,))


def kernel(user_id, item_id, category, brand,
           W_user_id, W_item_id, W_category, W_brand):
    out_a = _kernel_a(user_id, category, brand,
                      W_user_id, W_category, W_brand)
    out_b = _kernel_b(item_id, W_item_id)
    return (out_a + out_b).reshape(_B, _DTOT)
